# baseline (speedup 1.0000x reference)
import functools

import jax
import jax.numpy as jnp
from jax import lax
from jax.experimental import pallas as pl
from jax.experimental.pallas import tpu as pltpu
from jax.experimental.pallas import tpu_sc as plsc

N = 50000
E = 800000
H = 64
P = 128
B = 256

N_PAD = 51200
NW = 32
E_W = 25600
E_PAD = E_W * NW
CH = 128
G_ITERS = E_W // (8 * CH)
DUMMY = 50000
BN = 3200
UB = 2048
S_ROWS = N_PAD // 16
S_EDGE = E_PAD // 16


def _silu(v):
    return v * (1.0 / (1.0 + jnp.exp(-v)))


def _prep_body(z_ref, x_ref, emb_ref, win_ref, bin_ref, wa_ref, wb_ref,
               be1_ref, h0_ref, ta_ref, tb_ref):
    z = z_ref[...]
    oh = (z == lax.broadcasted_iota(jnp.int32, (BN, 128), 1)).astype(jnp.float32)
    t = emb_ref[...] @ win_ref[...] + bin_ref[...]
    h0 = oh @ t
    a = h0 @ wa_ref[...] + be1_ref[...]
    b = h0 @ wb_ref[...]
    xblk = x_ref[...]
    z48 = jnp.zeros((BN, 48), jnp.float32)
    h0_ref[...] = h0
    ta_ref[...] = jnp.concatenate([a, xblk, z48], axis=1)
    tb_ref[...] = jnp.concatenate([b, xblk, z48], axis=1)


def _prep(zp, xp, embp, W_in, b_in, wa, wb, be1):
    full = lambda s: pl.BlockSpec(s, lambda i: (0, 0))
    return pl.pallas_call(
        _prep_body,
        grid=(N_PAD // BN,),
        in_specs=[
            pl.BlockSpec((BN, 1), lambda i: (i, 0)),
            pl.BlockSpec((BN, 16), lambda i: (i, 0)),
            full((128, H)), full((H, H)), full((1, H)),
            full((H, H)), full((H, H)), full((1, H)),
        ],
        out_specs=[
            pl.BlockSpec((BN, H), lambda i: (i, 0)),
            pl.BlockSpec((BN, 128), lambda i: (i, 0)),
            pl.BlockSpec((BN, 128), lambda i: (i, 0)),
        ],
        out_shape=[
            jax.ShapeDtypeStruct((N_PAD, H), jnp.float32),
            jax.ShapeDtypeStruct((N_PAD, 128), jnp.float32),
            jax.ShapeDtypeStruct((N_PAD, 128), jnp.float32),
        ],
    )(zp, xp, embp, W_in, b_in, wa, wb, be1)


def _gather_body(ta_hbm, tb_hbm, rowi_hbm, coli_hbm, wsq_hbm, u_hbm,
                 idxa, idxb, bufa, bufb, ubuf0, ubuf1, wv, sga0, sga1,
                 sgb0, sgb1, su):
    wid = lax.axis_index("s") * 2 + lax.axis_index("c")
    base = wid * E_W

    pltpu.sync_copy(wsq_hbm, wv)
    wk = [wv[0, pl.ds(16 * k, 16)] for k in range(4)]

    def compute_chunk(buf_a, buf_b, ub, half):
        def pair(p):
            urow = half * 64 + p
            for sub in range(2):
                e = 2 * p + sub
                xa = buf_a[e, pl.ds(64, 16)]
                xb = buf_b[e, pl.ds(64, 16)]
                dx = xa - xb
                sq = jnp.sum(dx * dx)
                sqv = jnp.full((16,), sq, jnp.float32)
                for k in range(4):
                    pre = (buf_a[e, pl.ds(16 * k, 16)]
                           + buf_b[e, pl.ds(16 * k, 16)] + sqv * wk[k])
                    ub[urow, pl.ds(sub * 64 + 16 * k, 16)] = pre

        plsc.parallel_loop(0, 64, 1, unroll=2)(pair)

    def gathers(c, slot):
        da = pltpu.async_copy(
            ta_hbm.at[idxa.at[c]], bufa.at[pl.ds(slot * CH, CH)],
            sga0 if slot == 0 else sga1)
        db = pltpu.async_copy(
            tb_hbm.at[idxb.at[c]], bufb.at[pl.ds(slot * CH, CH)],
            sgb0 if slot == 0 else sgb1)
        return da, db

    def body(g, carry):
        off = pl.multiple_of(base + g * (8 * CH), 8 * CH)
        coff = pl.multiple_of(off // CH, 8)
        pltpu.sync_copy(rowi_hbm.at[pl.ds(coff, 8)], idxa)
        pltpu.sync_copy(coli_hbm.at[pl.ds(coff, 8)], idxb)
        d = gathers(0, 0)

        for c in range(8):
            ub = ubuf0 if (c // 2) % 2 == 0 else ubuf1
            @pl.when(jnp.logical_or(g > 0, c >= 4))
            def _(c=c, ub=ub):
                if c % 2 == 0:
                    pltpu.make_async_copy(
                        ub, u_hbm.at[pl.ds(pl.multiple_of(off // 2, CH), CH)],
                        su).wait()

            dn = gathers(c + 1, (c + 1) % 2) if c < 7 else None
            d[0].wait()
            d[1].wait()
            compute_chunk(bufa.at[pl.ds((c % 2) * CH, CH)],
                          bufb.at[pl.ds((c % 2) * CH, CH)], ub, c % 2)
            d = dn
            if c % 2 == 1:
                uoff = pl.multiple_of(off // 2 + (c // 2) * CH, CH)
                pltpu.async_copy(ub, u_hbm.at[pl.ds(uoff, CH)], su)
        return carry

    lax.fori_loop(0, G_ITERS, body, 0)
    for ub in (ubuf0, ubuf1):
        pltpu.make_async_copy(
            ub, u_hbm.at[pl.ds(pl.multiple_of(base // 2, CH), CH)], su
        ).wait()


_gather_call = functools.partial(
    pl.kernel,
    out_type=[
        jax.ShapeDtypeStruct((E_PAD // 2, 128), jnp.float32),
    ],
    mesh=plsc.VectorSubcoreMesh(core_axis_name="c", subcore_axis_name="s"),
    scratch_types=[
        pltpu.VMEM((8, CH), jnp.int32),
        pltpu.VMEM((8, CH), jnp.int32),
        pltpu.VMEM((2 * CH, 128), jnp.float32),
        pltpu.VMEM((2 * CH, 128), jnp.float32),
        pltpu.VMEM((CH, 128), jnp.float32),
        pltpu.VMEM((CH, 128), jnp.float32),
        pltpu.VMEM((8, 128), jnp.float32),
        pltpu.SemaphoreType.DMA,
        pltpu.SemaphoreType.DMA,
        pltpu.SemaphoreType.DMA,
        pltpu.SemaphoreType.DMA,
        pltpu.SemaphoreType.DMA,
    ],
    compiler_params=pltpu.CompilerParams(needs_layout_passes=False),
)(_gather_body)


def _edge_body(u_ref, w2_ref, b2_ref, mlo_ref, mhi_ref):
    u = _silu(u_ref[...])
    m2 = _silu(u @ w2_ref[...] + b2_ref[...])
    m = jnp.concatenate([m2[:, :H], m2[:, H:]], axis=0)
    qw = UB // 2
    mlo_ref[...] = jnp.concatenate(
        [m[q * qw:(q + 1) * qw, :32] for q in range(4)], axis=1)
    mhi_ref[...] = jnp.concatenate(
        [m[q * qw:(q + 1) * qw, 32:] for q in range(4)], axis=1)


def _edge(u, W2, b2):
    full = lambda s: pl.BlockSpec(s, lambda i: (0, 0))
    return pl.pallas_call(
        _edge_body,
        grid=((E_PAD // 2) // UB,),
        in_specs=[
            pl.BlockSpec((UB, 128), lambda i: (i, 0)),
            full((128, 128)), full((1, 128)),
        ],
        out_specs=[
            pl.BlockSpec((UB // 2, 128), lambda i: (i, 0)),
            pl.BlockSpec((UB // 2, 128), lambda i: (i, 0)),
        ],
        out_shape=[
            jax.ShapeDtypeStruct((E_PAD // 4, 128), jnp.float32),
            jax.ShapeDtypeStruct((E_PAD // 4, 128), jnp.float32),
        ],
    )(u, W2, b2)


def _scatter_body(mlo_hbm, mhi_hbm, rowi_hbm, zer_hbm, alo_hbm, ahi_hbm,
                  acc, idx, mbuf, sem):
    c = lax.axis_index("c")
    s = lax.axis_index("s")
    roff = pl.multiple_of(s * S_ROWS, S_ROWS)
    pltpu.sync_copy(zer_hbm, acc.at[pl.ds(roff, S_ROWS)])
    plsc.subcore_barrier()

    base = s * S_EDGE

    def body(g, carry):
        off = pl.multiple_of(base + g * (8 * CH), 8 * CH)
        coff = pl.multiple_of(off // CH, 8)
        pltpu.sync_copy(rowi_hbm.at[pl.ds(coff, 8)], idx)

        for half in range(2):
            hoff = pl.multiple_of(off + half * 4 * CH, 4 * CH)

            @pl.when(c == 0)
            def _():
                pltpu.sync_copy(mlo_hbm.at[pl.ds(hoff, 4 * CH)], mbuf)

            @pl.when(c == 1)
            def _():
                pltpu.sync_copy(mhi_hbm.at[pl.ds(hoff, 4 * CH)], mbuf)

            descs = []
            for j in range(4):
                descs.append(pltpu.async_copy(
                    mbuf.at[pl.ds(j * CH, CH)], acc.at[idx.at[half * 4 + j]],
                    sem, add=True))
            for d in descs:
                d.wait()
        return carry

    lax.fori_loop(0, S_EDGE // (8 * CH), body, 0)
    plsc.subcore_barrier()

    @pl.when(c == 0)
    def _():
        pltpu.sync_copy(acc.at[pl.ds(roff, S_ROWS)],
                        alo_hbm.at[pl.ds(roff, S_ROWS)])

    @pl.when(c == 1)
    def _():
        pltpu.sync_copy(acc.at[pl.ds(roff, S_ROWS)],
                        ahi_hbm.at[pl.ds(roff, S_ROWS)])


_scatter_call = functools.partial(
    pl.kernel,
    out_type=[
        jax.ShapeDtypeStruct((N_PAD, 32), jnp.float32),
        jax.ShapeDtypeStruct((N_PAD, 32), jnp.float32),
    ],
    mesh=plsc.VectorSubcoreMesh(core_axis_name="c", subcore_axis_name="s"),
    scratch_types=[
        pltpu.VMEM_SHARED((N_PAD, 32), jnp.float32),
        pltpu.VMEM((8, CH), jnp.int32),
        pltpu.VMEM((4 * CH, 32), jnp.float32),
        pltpu.SemaphoreType.DMA,
    ],
    compiler_params=pltpu.CompilerParams(use_tc_tiling_on_sc=False),
)(_scatter_body)


def _node_body(h0_ref, alo_ref, ahi_ref, bi_ref, wn1_ref, bn1_ref, wn2_ref,
               bn2_ref, wo_ref, bo_ref, wp_ref, bp_ref, z_ref, acc_ref):
    i = pl.program_id(0)
    h = h0_ref[...]
    nf = jnp.concatenate([h, alo_ref[...], ahi_ref[...]], axis=1)
    t = _silu(nf @ wn1_ref[...] + bn1_ref[...])
    h2 = h + (t @ wn2_ref[...] + bn2_ref[...])
    h3 = h2 @ wo_ref[...] + bo_ref[...]
    bi = bi_ref[...]
    oh = (bi == lax.broadcasted_iota(jnp.int32, (BN, B), 1)).astype(jnp.float32)
    hext = jnp.concatenate(
        [h3, jnp.ones((BN, 1), jnp.float32), jnp.zeros((BN, 63), jnp.float32)],
        axis=1)
    part = lax.dot_general(oh, hext, (((0,), (0,)), ((), ())))

    @pl.when(i == 0)
    def _():
        acc_ref[...] = part

    @pl.when(i > 0)
    def _():
        acc_ref[...] = acc_ref[...] + part

    @pl.when(i == pl.num_programs(0) - 1)
    def _():
        acc = acc_ref[...]
        mean = acc[:, :H] / jnp.clip(acc[:, H:H + 1], 1.0, None)
        z = mean @ wp_ref[...] + bp_ref[...]
        nrm = jnp.sqrt(jnp.sum(z * z, axis=1, keepdims=True))
        z_ref[...] = z / jnp.clip(nrm, 1e-12, None)


def _node(h0, alo, ahi, bip, W_n1, bn1, W_n2, bn2, W_out, bo, W_p, bp):
    full = lambda s: pl.BlockSpec(s, lambda i: (0, 0))
    return pl.pallas_call(
        _node_body,
        grid=(N_PAD // BN,),
        in_specs=[
            pl.BlockSpec((BN, H), lambda i: (i, 0)),
            pl.BlockSpec((BN, 32), lambda i: (i, 0)),
            pl.BlockSpec((BN, 32), lambda i: (i, 0)),
            pl.BlockSpec((BN, 1), lambda i: (i, 0)),
            full((2 * H, H)), full((1, H)), full((H, H)), full((1, H)),
            full((H, H)), full((1, H)), full((H, P)), full((1, P)),
        ],
        out_specs=pl.BlockSpec((B, P), lambda i: (0, 0)),
        out_shape=jax.ShapeDtypeStruct((B, P), jnp.float32),
        scratch_shapes=[pltpu.VMEM((B, P), jnp.float32)],
    )(h0, alo, ahi, bip, W_n1, bn1, W_n2, bn2, W_out, bo, W_p, bp)


def kernel(Z, x, edges, batch_idx, atom_emb, W_in, b_in, W_e1, b_e1, W_e2,
           b_e2, W_c1, b_c1, W_c2, b_c2, W_n1, b_n1, W_n2, b_n2, W_out,
           b_out, W_p, b_p):
    f32 = jnp.float32
    i32 = jnp.int32

    zp = jnp.zeros((N_PAD, 1), i32).at[:N, 0].set(Z.astype(i32))
    xp = jnp.zeros((N_PAD, 16), f32).at[:N, :3].set(x)
    embp = jnp.zeros((128, H), f32).at[:119].set(atom_emb)
    wa = W_e1[:H]
    wb = W_e1[H:2 * H]
    wsq = jnp.zeros((8, 128), f32).at[0, :H].set(W_e1[2 * H])
    W2 = (jnp.zeros((128, 128), f32)
          .at[:H, :H].set(W_e2).at[H:, H:].set(W_e2))
    b2 = jnp.concatenate([b_e2, b_e2]).reshape(1, 128)

    row = edges[0].astype(i32)
    col = edges[1].astype(i32)
    e_w = E // NW
    rowp_flat = (jnp.full((NW, E_W), DUMMY, i32)
                 .at[:, :e_w].set(row.reshape(NW, e_w)).reshape(-1))
    rowp = rowp_flat.reshape(E_PAD // CH, CH)
    colp = (jnp.full((NW, E_W), DUMMY, i32)
            .at[:, :e_w].set(col.reshape(NW, e_w)).reshape(E_PAD // CH, CH))
    rowp_k4 = (rowp_flat.reshape(E_PAD // (2 * UB), 2, UB // 2, 2)
               .transpose(0, 2, 3, 1).reshape(E_PAD // CH, CH))

    h0, ta, tb = _prep(zp, xp, embp, W_in, b_in.reshape(1, H), wa, wb,
                       b_e1.reshape(1, H))
    (u,) = _gather_call(ta, tb, rowp, colp, wsq)
    mlo4, mhi4 = _edge(u, W2, b2)
    mlo = mlo4.reshape(E_PAD, 32)
    mhi = mhi4.reshape(E_PAD, 32)
    zer = jnp.zeros((S_ROWS, 32), f32)
    alo, ahi = _scatter_call(mlo, mhi, rowp_k4, zer)

    bip = jnp.full((N_PAD, 1), -1, i32).at[:N, 0].set(batch_idx.astype(i32))
    z = _node(h0, alo, ahi, bip, W_n1, b_n1.reshape(1, H), W_n2,
              b_n2.reshape(1, H), W_out, b_out.reshape(1, H), W_p,
              b_p.reshape(1, P))
    return z


# speedup vs baseline: 1.1260x; 1.1260x over previous
import functools

import jax
import jax.numpy as jnp
from jax import lax
from jax.experimental import pallas as pl
from jax.experimental.pallas import tpu as pltpu
from jax.experimental.pallas import tpu_sc as plsc

N = 50000
E = 800000
H = 64
P = 128
B = 256

N_PAD = 51200
NW = 32
E_W = 25600
E_PAD = E_W * NW
CH = 128
G_ITERS = E_W // (8 * CH)
DUMMY = 50000
BN = 3200
UB = 512
NHALF = N_PAD // 2
ACC_R = 25728
T_ROWS = ACC_R // 16
T_EDGE = E_PAD // 16


def _silu(v):
    return v * (1.0 / (1.0 + jnp.exp(-v)))


def _prep_body(z_ref, x_ref, emb_ref, win_ref, bin_ref, wa_ref, wb_ref,
               be1_ref, h0_ref, ta_ref, tb_ref):
    z = z_ref[...]
    oh = (z == lax.broadcasted_iota(jnp.int32, (BN, 128), 1)).astype(jnp.float32)
    t = emb_ref[...] @ win_ref[...] + bin_ref[...]
    h0 = oh @ t
    a = h0 @ wa_ref[...] + be1_ref[...]
    b = h0 @ wb_ref[...]
    xblk = x_ref[...]
    z48 = jnp.zeros((BN, 48), jnp.float32)
    h0_ref[...] = h0
    ta_ref[...] = jnp.concatenate([a, xblk, z48], axis=1)
    tb_ref[...] = jnp.concatenate([b, xblk, z48], axis=1)


def _prep(zp, xp, embp, W_in, b_in, wa, wb, be1):
    full = lambda s: pl.BlockSpec(s, lambda i: (0, 0))
    return pl.pallas_call(
        _prep_body,
        grid=(N_PAD // BN,),
        in_specs=[
            pl.BlockSpec((BN, 1), lambda i: (i, 0)),
            pl.BlockSpec((BN, 16), lambda i: (i, 0)),
            full((128, H)), full((H, H)), full((1, H)),
            full((H, H)), full((H, H)), full((1, H)),
        ],
        out_specs=[
            pl.BlockSpec((BN, H), lambda i: (i, 0)),
            pl.BlockSpec((BN, 128), lambda i: (i, 0)),
            pl.BlockSpec((BN, 128), lambda i: (i, 0)),
        ],
        out_shape=[
            jax.ShapeDtypeStruct((N_PAD, H), jnp.float32),
            jax.ShapeDtypeStruct((N_PAD, 128), jnp.float32),
            jax.ShapeDtypeStruct((N_PAD, 128), jnp.float32),
        ],
    )(zp, xp, embp, W_in, b_in, wa, wb, be1)


def _gather_body(ta_hbm, tb_hbm, rowi_hbm, coli_hbm, wsq_hbm, u_hbm,
                 idxa, idxb, bufa, bufb, ubuf0, ubuf1, wv, sga0, sga1,
                 sgb0, sgb1, su):
    wid = lax.axis_index("s") * 2 + lax.axis_index("c")
    base = wid * E_W

    pltpu.sync_copy(wsq_hbm, wv)
    wk = [wv[0, pl.ds(16 * k, 16)] for k in range(4)]

    def compute_chunk(buf_a, buf_b, ub, half):
        def pair(p):
            urow = half * 64 + p
            for sub in range(2):
                e = 2 * p + sub
                xa = buf_a[e, pl.ds(64, 16)]
                xb = buf_b[e, pl.ds(64, 16)]
                dx = xa - xb
                sq = jnp.sum(dx * dx)
                sqv = jnp.full((16,), sq, jnp.float32)
                for k in range(4):
                    pre = (buf_a[e, pl.ds(16 * k, 16)]
                           + buf_b[e, pl.ds(16 * k, 16)] + sqv * wk[k])
                    ub[urow, pl.ds(sub * 64 + 16 * k, 16)] = pre

        plsc.parallel_loop(0, 64, 1, unroll=2)(pair)

    def gathers(c, slot):
        da = pltpu.async_copy(
            ta_hbm.at[idxa.at[c]], bufa.at[pl.ds(slot * CH, CH)],
            sga0 if slot == 0 else sga1)
        db = pltpu.async_copy(
            tb_hbm.at[idxb.at[c]], bufb.at[pl.ds(slot * CH, CH)],
            sgb0 if slot == 0 else sgb1)
        return da, db

    def body(g, carry):
        off = pl.multiple_of(base + g * (8 * CH), 8 * CH)
        coff = pl.multiple_of(off // CH, 8)
        pltpu.sync_copy(rowi_hbm.at[pl.ds(coff, 8)], idxa)
        pltpu.sync_copy(coli_hbm.at[pl.ds(coff, 8)], idxb)
        d = gathers(0, 0)

        for c in range(8):
            ub = ubuf0 if (c // 2) % 2 == 0 else ubuf1
            @pl.when(jnp.logical_or(g > 0, c >= 4))
            def _(c=c, ub=ub):
                if c % 2 == 0:
                    pltpu.make_async_copy(
                        ub, u_hbm.at[pl.ds(pl.multiple_of(off // 2, CH), CH)],
                        su).wait()

            dn = gathers(c + 1, (c + 1) % 2) if c < 7 else None
            d[0].wait()
            d[1].wait()
            compute_chunk(bufa.at[pl.ds((c % 2) * CH, CH)],
                          bufb.at[pl.ds((c % 2) * CH, CH)], ub, c % 2)
            d = dn
            if c % 2 == 1:
                uoff = pl.multiple_of(off // 2 + (c // 2) * CH, CH)
                pltpu.async_copy(ub, u_hbm.at[pl.ds(uoff, CH)], su)
        return carry

    lax.fori_loop(0, G_ITERS, body, 0)
    for ub in (ubuf0, ubuf1):
        pltpu.make_async_copy(
            ub, u_hbm.at[pl.ds(pl.multiple_of(base // 2, CH), CH)], su
        ).wait()


_gather_call = functools.partial(
    pl.kernel,
    out_type=[
        jax.ShapeDtypeStruct((E_PAD // 2, 128), jnp.float32),
    ],
    mesh=plsc.VectorSubcoreMesh(core_axis_name="c", subcore_axis_name="s"),
    scratch_types=[
        pltpu.VMEM((8, CH), jnp.int32),
        pltpu.VMEM((8, CH), jnp.int32),
        pltpu.VMEM((2 * CH, 128), jnp.float32),
        pltpu.VMEM((2 * CH, 128), jnp.float32),
        pltpu.VMEM((CH, 128), jnp.float32),
        pltpu.VMEM((CH, 128), jnp.float32),
        pltpu.VMEM((8, 128), jnp.float32),
        pltpu.SemaphoreType.DMA,
        pltpu.SemaphoreType.DMA,
        pltpu.SemaphoreType.DMA,
        pltpu.SemaphoreType.DMA,
        pltpu.SemaphoreType.DMA,
    ],
    compiler_params=pltpu.CompilerParams(needs_layout_passes=False),
)(_gather_body)


def _edge_body(u_ref, w2_ref, b2_ref, m_ref):
    u = _silu(u_ref[...])
    m_ref[...] = _silu(u @ w2_ref[...] + b2_ref[...])


def _edge(u, W2, b2):
    full = lambda s: pl.BlockSpec(s, lambda i: (0, 0))
    return pl.pallas_call(
        _edge_body,
        grid=((E_PAD // 2) // UB,),
        in_specs=[
            pl.BlockSpec((UB, 128), lambda i: (i, 0)),
            full((128, 128)), full((1, 128)),
        ],
        out_specs=pl.BlockSpec((UB, 128), lambda i: (i, 0)),
        out_shape=jax.ShapeDtypeStruct((E_PAD // 2, 128), jnp.float32),
    )(u, W2, b2)


def _scatter_body(m_hbm, rowi_hbm, zer_hbm, alo_hbm, ahi_hbm,
                  acc, idx, mbuf, sem):
    c = lax.axis_index("c")
    s = lax.axis_index("s")
    nbase = c * NHALF
    roff = pl.multiple_of(s * T_ROWS, 8)
    pltpu.sync_copy(zer_hbm, acc.at[pl.ds(roff, T_ROWS)])
    plsc.subcore_barrier()

    base = s * T_EDGE

    def body(g, carry):
        off = pl.multiple_of(base + g * (8 * CH), 8 * CH)
        coff = pl.multiple_of(off // CH, 8)
        pltpu.sync_copy(rowi_hbm.at[pl.ds(coff, 8)], idx)
        for j in range(8):
            for l in range(8):
                v = idx[j, pl.ds(16 * l, 16)]
                w = v - jnp.full((16,), nbase, jnp.int32)
                ok = jnp.logical_and(w >= 0, w < NHALF)
                idx[j, pl.ds(16 * l, 16)] = jnp.where(
                    ok, w, jnp.full((16,), NHALF, jnp.int32))
        for q in range(4):
            moff = pl.multiple_of(off + q * 2 * CH, 2 * CH)
            pltpu.sync_copy(m_hbm.at[pl.ds(moff, 2 * CH)], mbuf)
            d0 = pltpu.async_copy(
                mbuf.at[pl.ds(0, CH)], acc.at[idx.at[2 * q]], sem, add=True)
            d1 = pltpu.async_copy(
                mbuf.at[pl.ds(CH, CH)], acc.at[idx.at[2 * q + 1]], sem,
                add=True)
            d0.wait()
            d1.wait()
        return carry

    lax.fori_loop(0, T_EDGE // (8 * CH), body, 0)
    plsc.subcore_barrier()

    @pl.when(c == 0)
    def _():
        pltpu.sync_copy(acc.at[pl.ds(roff, T_ROWS)],
                        alo_hbm.at[pl.ds(roff, T_ROWS)])

    @pl.when(c == 1)
    def _():
        pltpu.sync_copy(acc.at[pl.ds(roff, T_ROWS)],
                        ahi_hbm.at[pl.ds(roff, T_ROWS)])


_scatter_call = functools.partial(
    pl.kernel,
    out_type=[
        jax.ShapeDtypeStruct((ACC_R, H), jnp.float32),
        jax.ShapeDtypeStruct((ACC_R, H), jnp.float32),
    ],
    mesh=plsc.VectorSubcoreMesh(core_axis_name="c", subcore_axis_name="s"),
    scratch_types=[
        pltpu.VMEM_SHARED((ACC_R, H), jnp.float32),
        pltpu.VMEM((8, CH), jnp.int32),
        pltpu.VMEM((2 * CH, H), jnp.float32),
        pltpu.SemaphoreType.DMA,
    ],
    compiler_params=pltpu.CompilerParams(use_tc_tiling_on_sc=False,
                                         needs_layout_passes=False),
)(_scatter_body)


def _node_body(h0_ref, agg_ref, bi_ref, wn1_ref, bn1_ref, wn2_ref,
               bn2_ref, wo_ref, bo_ref, wp_ref, bp_ref, z_ref, acc_ref):
    i = pl.program_id(0)
    h = h0_ref[...]
    nf = jnp.concatenate([h, agg_ref[...]], axis=1)
    t = _silu(nf @ wn1_ref[...] + bn1_ref[...])
    h2 = h + (t @ wn2_ref[...] + bn2_ref[...])
    h3 = h2 @ wo_ref[...] + bo_ref[...]
    bi = bi_ref[...]
    oh = (bi == lax.broadcasted_iota(jnp.int32, (BN, B), 1)).astype(jnp.float32)
    hext = jnp.concatenate(
        [h3, jnp.ones((BN, 1), jnp.float32), jnp.zeros((BN, 63), jnp.float32)],
        axis=1)
    part = lax.dot_general(oh, hext, (((0,), (0,)), ((), ())))

    @pl.when(i == 0)
    def _():
        acc_ref[...] = part

    @pl.when(i > 0)
    def _():
        acc_ref[...] = acc_ref[...] + part

    @pl.when(i == pl.num_programs(0) - 1)
    def _():
        acc = acc_ref[...]
        mean = acc[:, :H] / jnp.clip(acc[:, H:H + 1], 1.0, None)
        z = mean @ wp_ref[...] + bp_ref[...]
        nrm = jnp.sqrt(jnp.sum(z * z, axis=1, keepdims=True))
        z_ref[...] = z / jnp.clip(nrm, 1e-12, None)


def _node(h0, agg, bip, W_n1, bn1, W_n2, bn2, W_out, bo, W_p, bp):
    full = lambda s: pl.BlockSpec(s, lambda i: (0, 0))
    return pl.pallas_call(
        _node_body,
        grid=(N_PAD // BN,),
        in_specs=[
            pl.BlockSpec((BN, H), lambda i: (i, 0)),
            pl.BlockSpec((BN, H), lambda i: (i, 0)),
            pl.BlockSpec((BN, 1), lambda i: (i, 0)),
            full((2 * H, H)), full((1, H)), full((H, H)), full((1, H)),
            full((H, H)), full((1, H)), full((H, P)), full((1, P)),
        ],
        out_specs=pl.BlockSpec((B, P), lambda i: (0, 0)),
        out_shape=jax.ShapeDtypeStruct((B, P), jnp.float32),
        scratch_shapes=[pltpu.VMEM((B, P), jnp.float32)],
    )(h0, agg, bip, W_n1, bn1, W_n2, bn2, W_out, bo, W_p, bp)


def kernel(Z, x, edges, batch_idx, atom_emb, W_in, b_in, W_e1, b_e1, W_e2,
           b_e2, W_c1, b_c1, W_c2, b_c2, W_n1, b_n1, W_n2, b_n2, W_out,
           b_out, W_p, b_p):
    f32 = jnp.float32
    i32 = jnp.int32

    zp = jnp.zeros((N_PAD, 1), i32).at[:N, 0].set(Z.astype(i32))
    xp = jnp.zeros((N_PAD, 16), f32).at[:N, :3].set(x)
    embp = jnp.zeros((128, H), f32).at[:119].set(atom_emb)
    wa = W_e1[:H]
    wb = W_e1[H:2 * H]
    wsq = jnp.zeros((8, 128), f32).at[0, :H].set(W_e1[2 * H])
    W2 = (jnp.zeros((128, 128), f32)
          .at[:H, :H].set(W_e2).at[H:, H:].set(W_e2))
    b2 = jnp.concatenate([b_e2, b_e2]).reshape(1, 128)

    row = edges[0].astype(i32)
    col = edges[1].astype(i32)
    e_w = E // NW
    rowp = (jnp.full((NW, E_W), DUMMY, i32)
            .at[:, :e_w].set(row.reshape(NW, e_w)).reshape(E_PAD // CH, CH))
    colp = (jnp.full((NW, E_W), DUMMY, i32)
            .at[:, :e_w].set(col.reshape(NW, e_w)).reshape(E_PAD // CH, CH))

    h0, ta, tb = _prep(zp, xp, embp, W_in, b_in.reshape(1, H), wa, wb,
                       b_e1.reshape(1, H))
    (u,) = _gather_call(ta, tb, rowp, colp, wsq)
    m2 = _edge(u, W2, b2)
    m = m2.reshape(E_PAD, H)
    zer = jnp.zeros((T_ROWS, H), f32)
    alo, ahi = _scatter_call(m, rowp, zer)
    agg = jnp.concatenate([alo[:NHALF], ahi[:NHALF]], axis=0)

    bip = jnp.full((N_PAD, 1), -1, i32).at[:N, 0].set(batch_idx.astype(i32))
    z = _node(h0, agg, bip, W_n1, b_n1.reshape(1, H), W_n2,
              b_n2.reshape(1, H), W_out, b_out.reshape(1, H), W_p,
              b_p.reshape(1, P))
    return z


# speedup vs baseline: 1.2285x; 1.0911x over previous
import functools

import jax
import jax.numpy as jnp
from jax import lax
from jax.experimental import pallas as pl
from jax.experimental.pallas import tpu as pltpu
from jax.experimental.pallas import tpu_sc as plsc

N = 50000
E = 800000
H = 64
P = 128
B = 256

N_PAD = 51200
NW = 32
E_W = 25600
E_PAD = E_W * NW
CH = 128
G_ITERS = E_W // (8 * CH)
DUMMY = 50000
BN = 3200
UB = 1024
NHALF = N_PAD // 2
ACC_R = 25728
T_ROWS = ACC_R // 16
T_EDGE = E_PAD // 16


def _silu(v):
    return v * (1.0 / (1.0 + jnp.exp(-v)))


def _prep_body(z_ref, x_ref, emb_ref, win_ref, bin_ref, wa_ref, wb_ref,
               be1_ref, h0_ref, ta_ref, tb_ref):
    z = z_ref[...]
    oh = (z == lax.broadcasted_iota(jnp.int32, (BN, 128), 1)).astype(jnp.float32)
    t = emb_ref[...] @ win_ref[...] + bin_ref[...]
    h0 = oh @ t
    a = h0 @ wa_ref[...] + be1_ref[...]
    b = h0 @ wb_ref[...]
    xblk = x_ref[...]
    z48 = jnp.zeros((BN, 48), jnp.float32)
    h0_ref[...] = h0
    ta_ref[...] = jnp.concatenate([a, xblk, z48], axis=1)
    tb_ref[...] = jnp.concatenate([b, xblk, z48], axis=1)


def _prep(zp, xp, embp, W_in, b_in, wa, wb, be1):
    full = lambda s: pl.BlockSpec(s, lambda i: (0, 0))
    return pl.pallas_call(
        _prep_body,
        grid=(N_PAD // BN,),
        in_specs=[
            pl.BlockSpec((BN, 1), lambda i: (i, 0)),
            pl.BlockSpec((BN, 16), lambda i: (i, 0)),
            full((128, H)), full((H, H)), full((1, H)),
            full((H, H)), full((H, H)), full((1, H)),
        ],
        out_specs=[
            pl.BlockSpec((BN, H), lambda i: (i, 0)),
            pl.BlockSpec((BN, 128), lambda i: (i, 0)),
            pl.BlockSpec((BN, 128), lambda i: (i, 0)),
        ],
        out_shape=[
            jax.ShapeDtypeStruct((N_PAD, H), jnp.float32),
            jax.ShapeDtypeStruct((N_PAD, 128), jnp.float32),
            jax.ShapeDtypeStruct((N_PAD, 128), jnp.float32),
        ],
    )(zp, xp, embp, W_in, b_in, wa, wb, be1)


def _gather_body(ta_hbm, tb_hbm, rowi_hbm, coli_hbm, wsq_hbm, u_hbm,
                 idxa, idxb, bufa, bufb, ubuf0, ubuf1, wv, sga0, sga1,
                 sgb0, sgb1, su):
    wid = lax.axis_index("s") * 2 + lax.axis_index("c")
    base = wid * E_W

    pltpu.sync_copy(wsq_hbm, wv)
    wk = [wv[0, pl.ds(16 * k, 16)] for k in range(4)]

    def compute_chunk(buf_a, buf_b, ub, half):
        def pair(p):
            urow = half * 64 + p
            for sub in range(2):
                e = 2 * p + sub
                xa = buf_a[e, pl.ds(64, 16)]
                xb = buf_b[e, pl.ds(64, 16)]
                dx = xa - xb
                sq = jnp.sum(dx * dx)
                sqv = jnp.full((16,), sq, jnp.float32)
                for k in range(4):
                    pre = (buf_a[e, pl.ds(16 * k, 16)]
                           + buf_b[e, pl.ds(16 * k, 16)] + sqv * wk[k])
                    ub[urow, pl.ds(sub * 64 + 16 * k, 16)] = pre

        plsc.parallel_loop(0, 64, 1, unroll=2)(pair)

    def gathers(c, slot):
        da = pltpu.async_copy(
            ta_hbm.at[idxa.at[c]], bufa.at[pl.ds(slot * CH, CH)],
            sga0 if slot == 0 else sga1)
        db = pltpu.async_copy(
            tb_hbm.at[idxb.at[c]], bufb.at[pl.ds(slot * CH, CH)],
            sgb0 if slot == 0 else sgb1)
        return da, db

    def body(g, carry):
        off = pl.multiple_of(base + g * (8 * CH), 8 * CH)
        coff = pl.multiple_of(off // CH, 8)
        pltpu.sync_copy(rowi_hbm.at[pl.ds(coff, 8)], idxa)
        pltpu.sync_copy(coli_hbm.at[pl.ds(coff, 8)], idxb)
        d = gathers(0, 0)

        for c in range(8):
            ub = ubuf0 if (c // 2) % 2 == 0 else ubuf1
            @pl.when(jnp.logical_or(g > 0, c >= 4))
            def _(c=c, ub=ub):
                if c % 2 == 0:
                    pltpu.make_async_copy(
                        ub, u_hbm.at[pl.ds(pl.multiple_of(off // 2, CH), CH)],
                        su).wait()

            dn = gathers(c + 1, (c + 1) % 2) if c < 7 else None
            d[0].wait()
            d[1].wait()
            compute_chunk(bufa.at[pl.ds((c % 2) * CH, CH)],
                          bufb.at[pl.ds((c % 2) * CH, CH)], ub, c % 2)
            d = dn
            if c % 2 == 1:
                uoff = pl.multiple_of(off // 2 + (c // 2) * CH, CH)
                pltpu.async_copy(ub, u_hbm.at[pl.ds(uoff, CH)], su)
        return carry

    lax.fori_loop(0, G_ITERS, body, 0)
    for ub in (ubuf0, ubuf1):
        pltpu.make_async_copy(
            ub, u_hbm.at[pl.ds(pl.multiple_of(base // 2, CH), CH)], su
        ).wait()


_gather_call = functools.partial(
    pl.kernel,
    out_type=[
        jax.ShapeDtypeStruct((E_PAD // 2, 128), jnp.float32),
    ],
    mesh=plsc.VectorSubcoreMesh(core_axis_name="c", subcore_axis_name="s"),
    scratch_types=[
        pltpu.VMEM((8, CH), jnp.int32),
        pltpu.VMEM((8, CH), jnp.int32),
        pltpu.VMEM((2 * CH, 128), jnp.float32),
        pltpu.VMEM((2 * CH, 128), jnp.float32),
        pltpu.VMEM((CH, 128), jnp.float32),
        pltpu.VMEM((CH, 128), jnp.float32),
        pltpu.VMEM((8, 128), jnp.float32),
        pltpu.SemaphoreType.DMA,
        pltpu.SemaphoreType.DMA,
        pltpu.SemaphoreType.DMA,
        pltpu.SemaphoreType.DMA,
        pltpu.SemaphoreType.DMA,
    ],
    compiler_params=pltpu.CompilerParams(needs_layout_passes=False),
)(_gather_body)


def _edge_body(u_ref, w2_ref, b2_ref, m_ref):
    u = _silu(u_ref[...])
    m_ref[...] = _silu(u @ w2_ref[...] + b2_ref[...])


def _edge(u, W2, b2):
    full = lambda s: pl.BlockSpec(s, lambda i: (0, 0))
    return pl.pallas_call(
        _edge_body,
        grid=((E_PAD // 2) // UB,),
        in_specs=[
            pl.BlockSpec((UB, 128), lambda i: (i, 0)),
            full((128, 128)), full((1, 128)),
        ],
        out_specs=pl.BlockSpec((UB, 128), lambda i: (i, 0)),
        out_shape=jax.ShapeDtypeStruct((E_PAD // 2, 128), jnp.float32),
    )(u, W2, b2)


def _scatter_body(m_hbm, rowi_hbm, zer_hbm, alo_hbm, ahi_hbm,
                  acc, idx, mbuf, sem):
    c = lax.axis_index("c")
    s = lax.axis_index("s")
    nbase = c * NHALF
    roff = pl.multiple_of(s * T_ROWS, 8)
    pltpu.sync_copy(zer_hbm, acc.at[pl.ds(roff, T_ROWS)])
    plsc.subcore_barrier()

    base = s * T_EDGE

    def body(g, carry):
        off = pl.multiple_of(base + g * (8 * CH), 8 * CH)
        coff = pl.multiple_of(off // CH, 8)
        pltpu.sync_copy(rowi_hbm.at[pl.ds(coff, 8)], idx)
        for j in range(8):
            for l in range(8):
                v = idx[j, pl.ds(16 * l, 16)]
                w = v - jnp.full((16,), nbase, jnp.int32)
                ok = jnp.logical_and(w >= 0, w < NHALF)
                idx[j, pl.ds(16 * l, 16)] = jnp.where(
                    ok, w, jnp.full((16,), NHALF, jnp.int32))
        for q in range(4):
            moff = pl.multiple_of(off + q * 2 * CH, 2 * CH)
            pltpu.sync_copy(m_hbm.at[pl.ds(moff, 2 * CH)], mbuf)
            d0 = pltpu.async_copy(
                mbuf.at[pl.ds(0, CH)], acc.at[idx.at[2 * q]], sem, add=True)
            d1 = pltpu.async_copy(
                mbuf.at[pl.ds(CH, CH)], acc.at[idx.at[2 * q + 1]], sem,
                add=True)
            d0.wait()
            d1.wait()
        return carry

    lax.fori_loop(0, T_EDGE // (8 * CH), body, 0)
    plsc.subcore_barrier()

    @pl.when(c == 0)
    def _():
        pltpu.sync_copy(acc.at[pl.ds(roff, T_ROWS)],
                        alo_hbm.at[pl.ds(roff, T_ROWS)])

    @pl.when(c == 1)
    def _():
        pltpu.sync_copy(acc.at[pl.ds(roff, T_ROWS)],
                        ahi_hbm.at[pl.ds(roff, T_ROWS)])


_scatter_call = functools.partial(
    pl.kernel,
    out_type=[
        jax.ShapeDtypeStruct((ACC_R, H), jnp.float32),
        jax.ShapeDtypeStruct((ACC_R, H), jnp.float32),
    ],
    mesh=plsc.VectorSubcoreMesh(core_axis_name="c", subcore_axis_name="s"),
    scratch_types=[
        pltpu.VMEM_SHARED((ACC_R, H), jnp.float32),
        pltpu.VMEM((8, CH), jnp.int32),
        pltpu.VMEM((2 * CH, H), jnp.float32),
        pltpu.SemaphoreType.DMA,
    ],
    compiler_params=pltpu.CompilerParams(use_tc_tiling_on_sc=False,
                                         needs_layout_passes=False),
)(_scatter_body)


def _node_body(h0_ref, agg_ref, bi_ref, wn1_ref, bn1_ref, wn2_ref,
               bn2_ref, wo_ref, bo_ref, wp_ref, bp_ref, z_ref, acc_ref):
    i = pl.program_id(0)
    h = h0_ref[...]
    nf = jnp.concatenate([h, agg_ref[...]], axis=1)
    t = _silu(nf @ wn1_ref[...] + bn1_ref[...])
    h2 = h + (t @ wn2_ref[...] + bn2_ref[...])
    h3 = h2 @ wo_ref[...] + bo_ref[...]
    bi = bi_ref[...]
    oh = (bi == lax.broadcasted_iota(jnp.int32, (BN, B), 1)).astype(jnp.float32)
    hext = jnp.concatenate(
        [h3, jnp.ones((BN, 1), jnp.float32), jnp.zeros((BN, 63), jnp.float32)],
        axis=1)
    part = lax.dot_general(oh, hext, (((0,), (0,)), ((), ())))

    @pl.when(i == 0)
    def _():
        acc_ref[...] = part

    @pl.when(i > 0)
    def _():
        acc_ref[...] = acc_ref[...] + part

    @pl.when(i == pl.num_programs(0) - 1)
    def _():
        acc = acc_ref[...]
        mean = acc[:, :H] / jnp.clip(acc[:, H:H + 1], 1.0, None)
        z = mean @ wp_ref[...] + bp_ref[...]
        nrm = jnp.sqrt(jnp.sum(z * z, axis=1, keepdims=True))
        z_ref[...] = z / jnp.clip(nrm, 1e-12, None)


def _node(h0, agg, bip, W_n1, bn1, W_n2, bn2, W_out, bo, W_p, bp):
    full = lambda s: pl.BlockSpec(s, lambda i: (0, 0))
    return pl.pallas_call(
        _node_body,
        grid=(N_PAD // BN,),
        in_specs=[
            pl.BlockSpec((BN, H), lambda i: (i, 0)),
            pl.BlockSpec((BN, H), lambda i: (i, 0)),
            pl.BlockSpec((BN, 1), lambda i: (i, 0)),
            full((2 * H, H)), full((1, H)), full((H, H)), full((1, H)),
            full((H, H)), full((1, H)), full((H, P)), full((1, P)),
        ],
        out_specs=pl.BlockSpec((B, P), lambda i: (0, 0)),
        out_shape=jax.ShapeDtypeStruct((B, P), jnp.float32),
        scratch_shapes=[pltpu.VMEM((B, P), jnp.float32)],
    )(h0, agg, bip, W_n1, bn1, W_n2, bn2, W_out, bo, W_p, bp)


def kernel(Z, x, edges, batch_idx, atom_emb, W_in, b_in, W_e1, b_e1, W_e2,
           b_e2, W_c1, b_c1, W_c2, b_c2, W_n1, b_n1, W_n2, b_n2, W_out,
           b_out, W_p, b_p):
    f32 = jnp.float32
    i32 = jnp.int32

    zp = jnp.zeros((N_PAD, 1), i32).at[:N, 0].set(Z.astype(i32))
    xp = jnp.zeros((N_PAD, 16), f32).at[:N, :3].set(x)
    embp = jnp.zeros((128, H), f32).at[:119].set(atom_emb)
    wa = W_e1[:H]
    wb = W_e1[H:2 * H]
    wsq = jnp.zeros((8, 128), f32).at[0, :H].set(W_e1[2 * H])
    W2 = (jnp.zeros((128, 128), f32)
          .at[:H, :H].set(W_e2).at[H:, H:].set(W_e2))
    b2 = jnp.concatenate([b_e2, b_e2]).reshape(1, 128)

    row = edges[0].astype(i32)
    col = edges[1].astype(i32)
    e_w = E // NW
    rowp = (jnp.full((NW, E_W), DUMMY, i32)
            .at[:, :e_w].set(row.reshape(NW, e_w)).reshape(E_PAD // CH, CH))
    colp = (jnp.full((NW, E_W), DUMMY, i32)
            .at[:, :e_w].set(col.reshape(NW, e_w)).reshape(E_PAD // CH, CH))

    h0, ta, tb = _prep(zp, xp, embp, W_in, b_in.reshape(1, H), wa, wb,
                       b_e1.reshape(1, H))
    (u,) = _gather_call(ta, tb, rowp, colp, wsq)
    m2 = _edge(u, W2, b2)
    m = m2.reshape(E_PAD, H)
    zer = jnp.zeros((T_ROWS, H), f32)
    alo, ahi = _scatter_call(m, rowp, zer)
    agg = jnp.concatenate([alo[:NHALF], ahi[:NHALF]], axis=0)

    bip = jnp.full((N_PAD, 1), -1, i32).at[:N, 0].set(batch_idx.astype(i32))
    z = _node(h0, agg, bip, W_n1, b_n1.reshape(1, H), W_n2,
              b_n2.reshape(1, H), W_out, b_out.reshape(1, H), W_p,
              b_p.reshape(1, P))
    return z
